# Initial kernel scaffold; baseline (speedup 1.0000x reference)
import jax
import jax.numpy as jnp
from jax.experimental import pallas as pl


def kernel(ent_attr_ids_cont, vals_cont, days_cont, ent_attr_ids_categ, vocab_ids_categ, days_categ, pair_emb, categ_val_emb):
    raise NotImplementedError("write your pallas kernel here")


# speedup vs baseline: 10.6325x; 10.6325x over previous
import functools

import jax
import jax.numpy as jnp
from jax import lax
from jax.experimental import pallas as pl
from jax.experimental.pallas import tpu as pltpu
from jax.experimental.pallas import tpu_sc as plsc

P = 16
V = 16
D = 16
N_CONT = 262144
N_CATEG = 262144

NC = 2
NS = 16
NW = NC * NS
ROWS_W = N_CONT // NW
CHUNK = 2048
NCHUNK = ROWS_W // CHUNK
MB = CHUNK // 16

TWOPI = 6.283185307179586
INV2PI = 1.0 / TWOPI

SIN_C = (0.9999791148943297, -0.1666240153829831, 0.00830884993122673,
         -0.00019263169952744158, 2.147049615625063e-06)
COS_C = (0.9999992107412203, -0.4999942131500665, 0.04165977758594538,
         -0.0013858789204833017, 2.4202932054760706e-05,
         -2.1972921876445284e-07)

INV17 = tuple(10000.0 ** (-(2 * j) / 17.0) for j in range(9))
INV16 = tuple(10000.0 ** (-(2 * j) / 16.0) for j in range(8))


def _range_reduce(ang):
    q = (ang * INV2PI + 0.5).astype(jnp.int32).astype(jnp.float32)
    return ang - q * TWOPI


def _sin_poly(r, r2):
    s = jnp.float32(SIN_C[-1])
    for c in SIN_C[-2::-1]:
        s = s * r2 + c
    return s * r


def _cos_poly(r2):
    c = jnp.float32(COS_C[-1])
    for cc in COS_C[-2::-1]:
        c = c * r2 + cc
    return c


def _splat(k):
    return jnp.full((16,), k, jnp.int32)


def _embed_body(pc_hbm, vals_hbm, dc_hbm, pg_hbm, vg_hbm, dg_hbm,
                pair_hbm, valtab_hbm, outc_hbm, outg_hbm,
                pair_v, valtab_v, pc_v, vals_v, dc_v, pg_v, vg_v, dg_v,
                outc_v, outg_v):
    wid = lax.axis_index("s") * NC + lax.axis_index("c")
    base = wid * ROWS_W

    pltpu.sync_copy(pair_hbm, pair_v)
    pltpu.sync_copy(valtab_hbm, valtab_v)

    def chunk_body(chunk, carry):
        off = base + chunk * CHUNK
        pltpu.sync_copy(pc_hbm.at[pl.ds(off, CHUNK)], pc_v)
        pltpu.sync_copy(vals_hbm.at[pl.ds(off, CHUNK)], vals_v)
        pltpu.sync_copy(dc_hbm.at[pl.ds(off, CHUNK)], dc_v)
        pltpu.sync_copy(pg_hbm.at[pl.ds(off, CHUNK)], pg_v)
        pltpu.sync_copy(vg_hbm.at[pl.ds(off, CHUNK)], vg_v)
        pltpu.sync_copy(dg_hbm.at[pl.ds(off, CHUNK)], dg_v)

        def cont_mb(m, c2):
            sl = pl.ds(m * 16, 16)
            d_f = dc_v[sl].astype(jnp.float32)
            pb = pc_v[sl] * D
            rowsb = lax.iota(jnp.int32, 16) * 17 + m * (16 * 17)
            for j in range(8):
                r = _range_reduce(d_f * INV17[j])
                r2 = r * r
                s = _sin_poly(r, r2)
                co = _cos_poly(r2)
                ge = plsc.load_gather(pair_v, [pb + (2 * j)])
                go = plsc.load_gather(pair_v, [pb + (2 * j + 1)])
                plsc.store_scatter(outc_v, [rowsb + (2 * j)], ge + s)
                plsc.store_scatter(outc_v, [rowsb + (2 * j + 1)], go + co)
            r = _range_reduce(d_f * INV17[8])
            s = _sin_poly(r, r * r)
            plsc.store_scatter(outc_v, [rowsb + 16], vals_v[sl] + s)
            return c2

        lax.fori_loop(0, MB, cont_mb, 0)

        def categ_mb(m, c2):
            sl = pl.ds(m * 16, 16)
            d_f = dg_v[sl].astype(jnp.float32)
            p_i = pg_v[sl]
            pb = p_i * D
            vb = pb * V + vg_v[sl] * D
            rowsb = lax.iota(jnp.int32, 16) * 16 + m * (16 * 16)
            for j in range(8):
                r = _range_reduce(d_f * INV16[j])
                r2 = r * r
                s = _sin_poly(r, r2)
                co = _cos_poly(r2)
                pe_ = plsc.load_gather(pair_v, [pb + (2 * j)])
                po_ = plsc.load_gather(pair_v, [pb + (2 * j + 1)])
                ve_ = plsc.load_gather(valtab_v, [vb + (2 * j)])
                vo_ = plsc.load_gather(valtab_v, [vb + (2 * j + 1)])
                plsc.store_scatter(outg_v, [rowsb + (2 * j)], pe_ + ve_ + s)
                plsc.store_scatter(outg_v, [rowsb + (2 * j + 1)], po_ + vo_ + co)
            return c2

        lax.fori_loop(0, MB, categ_mb, 0)

        pltpu.sync_copy(outc_v, outc_hbm.at[pl.ds(off * 17, CHUNK * 17)])
        pltpu.sync_copy(outg_v, outg_hbm.at[pl.ds(off * 16, CHUNK * 16)])
        return carry

    lax.fori_loop(0, NCHUNK, chunk_body, 0)


@jax.jit
def kernel(ent_attr_ids_cont, vals_cont, days_cont,
           ent_attr_ids_categ, vocab_ids_categ, days_categ,
           pair_emb, categ_val_emb):
    mesh = plsc.VectorSubcoreMesh(core_axis_name="c", subcore_axis_name="s")
    f = pl.kernel(
        _embed_body,
        out_type=(jax.ShapeDtypeStruct((N_CONT * 17,), jnp.float32),
                  jax.ShapeDtypeStruct((N_CATEG * 16,), jnp.float32)),
        mesh=mesh,
        compiler_params=pltpu.CompilerParams(needs_layout_passes=False),
        scratch_types=[
            pltpu.VMEM((P * D,), jnp.float32),
            pltpu.VMEM((P * V * D,), jnp.float32),
            pltpu.VMEM((CHUNK,), jnp.int32),
            pltpu.VMEM((CHUNK,), jnp.float32),
            pltpu.VMEM((CHUNK,), jnp.int32),
            pltpu.VMEM((CHUNK,), jnp.int32),
            pltpu.VMEM((CHUNK,), jnp.int32),
            pltpu.VMEM((CHUNK,), jnp.int32),
            pltpu.VMEM((CHUNK * 17,), jnp.float32),
            pltpu.VMEM((CHUNK * 16,), jnp.float32),
        ],
    )
    outc, outg = f(ent_attr_ids_cont.astype(jnp.int32), vals_cont,
                   days_cont.astype(jnp.int32),
                   ent_attr_ids_categ.astype(jnp.int32),
                   vocab_ids_categ.astype(jnp.int32),
                   days_categ.astype(jnp.int32),
                   pair_emb.reshape(P * D), categ_val_emb.reshape(P * V * D))
    return outc.reshape(N_CONT, 17), outg.reshape(N_CATEG, 16)
